# baseline (speedup 1.0000x reference)
import jax
import jax.numpy as jnp
from jax import lax
from jax.experimental import pallas as pl
from jax.experimental.pallas import tpu as pltpu
from jax.experimental.pallas import tpu_sc as plsc

_NC = 2
_NS = 16
_NW = _NC * _NS
_L = 200
_B = 4096
_EMB = 128
_LBL = 100
_EPW = _B // _NW
_HALF = _L // 2
_VEC = 16
_KV = _EMB // _VEC
_NBUF = 4
_G0 = 96
_G1 = _L - _G0
_OUT_TILE = 8
_SCALE = 1.0 / _L


def _sc_pool_body(idx_hbm, table_hbm, out_hbm, idx_v,
                  rows0, rows1, rows2, rows3, outst,
                  sem0, sem1, sem2, sem3, out_sem):
    bufs = (rows0, rows1, rows2, rows3)
    sems = (sem0, sem1, sem2, sem3)
    wid = lax.axis_index("s") * _NC + lax.axis_index("c")
    base = pl.multiple_of(wid * _EPW, _EPW)
    nwords = _EPW * _L
    pltpu.sync_copy(
        idx_hbm.at[pl.ds(pl.multiple_of(wid * nwords, 8), nwords)], idx_v)

    _SPLITS = ((0, 48), (48, 48), (96, 48), (144, 56))

    def issue(e, buf, sem):
        for s, n in _SPLITS:
            off = pl.multiple_of(e * _L + s, 8)
            pltpu.async_copy(table_hbm.at[idx_v.at[pl.ds(off, n)]],
                             buf.at[pl.ds(s, n)], sem)

    def drain(buf, sem):
        for s, n in _SPLITS:
            pltpu.make_async_copy(table_hbm.at[idx_v.at[pl.ds(0, n)]],
                                  buf.at[pl.ds(s, n)], sem).wait()

    def consume(e, buf):
        zeros = tuple(jnp.zeros((_VEC,), jnp.float32) for _ in range(_KV))

        @plsc.parallel_loop(0, _HALF, unroll=2, carry=zeros)
        def acc(l, acc):
            acc = list(acc)
            for h in range(2):
                for k in range(_KV):
                    acc[k] = acc[k] + buf[l + _HALF * h,
                                          pl.ds(k * _VEC, _VEC)]
            return tuple(acc)
        m = lax.rem(e, _OUT_TILE)

        @pl.when(jnp.logical_and(m == 0, e >= _OUT_TILE))
        def _drain_prev_flush():
            pltpu.make_async_copy(out_hbm.at[pl.ds(0, _OUT_TILE)], outst,
                                  out_sem).wait()

        for k in range(_KV):
            outst[m, pl.ds(k * _VEC, _VEC)] = acc[k] * _SCALE

        @pl.when(m == _OUT_TILE - 1)
        def _flush():
            start = pl.multiple_of(base + e - (_OUT_TILE - 1), _OUT_TILE)
            pltpu.async_copy(outst, out_hbm.at[pl.ds(start, _OUT_TILE)],
                             out_sem)

    for q in range(_NBUF):
        issue(q, bufs[q], sems[q])

    def step(g, carry):
        e0 = _NBUF * g
        for q in range(_NBUF):
            e = e0 + q
            drain(bufs[q], sems[q])
            consume(e, bufs[q])

            @pl.when(e + _NBUF < _EPW)
            def _():
                issue(e + _NBUF, bufs[q], sems[q])
        return carry

    lax.fori_loop(0, _EPW // _NBUF, step, jnp.int32(0))
    pltpu.make_async_copy(out_hbm.at[pl.ds(0, _OUT_TILE)], outst,
                          out_sem).wait()


def _head_body(z_ref, w_ref, b_ref, o_ref):
    o_ref[...] = jax.nn.sigmoid(
        jnp.dot(z_ref[...], w_ref[...], preferred_element_type=jnp.float32)
        + b_ref[...])


def kernel(inputs, table, W, b):
    idx_flat = jnp.transpose(inputs).reshape(-1)
    mesh = plsc.VectorSubcoreMesh(core_axis_name="c", subcore_axis_name="s")
    pooled = pl.kernel(
        _sc_pool_body,
        mesh=mesh,
        out_type=jax.ShapeDtypeStruct((_B, _EMB), jnp.float32),
        scratch_types=[
            pltpu.VMEM((_EPW * _L,), jnp.int32),
        ] + [pltpu.VMEM((2 * _HALF, _EMB), jnp.float32)] * _NBUF + [
            pltpu.VMEM((_OUT_TILE, _EMB), jnp.float32),
        ] + [pltpu.SemaphoreType.DMA] * (_NBUF + 1),
    )(idx_flat, table)
    out = pl.pallas_call(
        _head_body,
        out_shape=jax.ShapeDtypeStruct((_B, _LBL), jnp.float32),
    )(pooled, W, jnp.reshape(b, (1, _LBL)))
    return out


# speedup vs baseline: 1.0011x; 1.0011x over previous
import jax
import jax.numpy as jnp
from jax import lax
from jax.experimental import pallas as pl
from jax.experimental.pallas import tpu as pltpu
from jax.experimental.pallas import tpu_sc as plsc

_NC = 2
_NS = 16
_NW = _NC * _NS
_L = 200
_B = 4096
_EMB = 128
_LBL = 100
_EPW = _B // _NW
_HALF = _L // 2
_VEC = 16
_KV = _EMB // _VEC
_NBUF = 4
_G0 = 96
_G1 = _L - _G0
_OUT_TILE = 8
_SCALE = 1.0 / _L


def _sc_pool_body(idx_hbm, table_hbm, out_hbm, idx_v,
                  rows0, rows1, rows2, rows3, outst,
                  sem0, sem1, sem2, sem3, out_sem):
    bufs = (rows0, rows1, rows2, rows3)
    sems = (sem0, sem1, sem2, sem3)
    wid = lax.axis_index("s") * _NC + lax.axis_index("c")
    base = pl.multiple_of(wid * _EPW, _EPW)
    nwords = _EPW * _L
    pltpu.sync_copy(
        idx_hbm.at[pl.ds(pl.multiple_of(wid * nwords, 8), nwords)], idx_v)

    _SPLITS = ((0, _G0), (_G0, _G1))

    def issue(e, buf, sem):
        for s, n in _SPLITS:
            off = pl.multiple_of(e * _L + s, 8)
            pltpu.async_copy(table_hbm.at[idx_v.at[pl.ds(off, n)]],
                             buf.at[pl.ds(s, n)], sem)

    def drain(buf, sem):
        for s, n in _SPLITS:
            pltpu.make_async_copy(table_hbm.at[idx_v.at[pl.ds(0, n)]],
                                  buf.at[pl.ds(s, n)], sem).wait()

    def consume(e, buf):
        zeros = tuple(jnp.zeros((_VEC,), jnp.float32) for _ in range(_KV))

        @plsc.parallel_loop(0, _HALF, unroll=2, carry=zeros)
        def acc(l, acc):
            acc = list(acc)
            for h in range(2):
                for k in range(_KV):
                    acc[k] = acc[k] + buf[l + _HALF * h,
                                          pl.ds(k * _VEC, _VEC)]
            return tuple(acc)
        m = lax.rem(e, _OUT_TILE)

        @pl.when(jnp.logical_and(m == 0, e >= _OUT_TILE))
        def _drain_prev_flush():
            pltpu.make_async_copy(out_hbm.at[pl.ds(0, _OUT_TILE)], outst,
                                  out_sem).wait()

        for k in range(_KV):
            outst[m, pl.ds(k * _VEC, _VEC)] = acc[k] * _SCALE

        @pl.when(m == _OUT_TILE - 1)
        def _flush():
            start = pl.multiple_of(base + e - (_OUT_TILE - 1), _OUT_TILE)
            pltpu.async_copy(outst, out_hbm.at[pl.ds(start, _OUT_TILE)],
                             out_sem)

    for q in range(_NBUF):
        issue(q, bufs[q], sems[q])

    def step(g, carry):
        e0 = _NBUF * g
        for q in range(_NBUF):
            e = e0 + q
            drain(bufs[q], sems[q])
            consume(e, bufs[q])

            @pl.when(e + _NBUF < _EPW)
            def _():
                issue(e + _NBUF, bufs[q], sems[q])
        return carry

    lax.fori_loop(0, _EPW // _NBUF, step, jnp.int32(0))
    pltpu.make_async_copy(out_hbm.at[pl.ds(0, _OUT_TILE)], outst,
                          out_sem).wait()


def _head_body(z_ref, w_ref, b_ref, o_ref):
    o_ref[...] = jax.nn.sigmoid(
        jnp.dot(z_ref[...], w_ref[...], preferred_element_type=jnp.float32)
        + b_ref[...])


def kernel(inputs, table, W, b):
    idx_flat = jnp.transpose(inputs).reshape(-1)
    mesh = plsc.VectorSubcoreMesh(core_axis_name="c", subcore_axis_name="s")
    pooled = pl.kernel(
        _sc_pool_body,
        mesh=mesh,
        out_type=jax.ShapeDtypeStruct((_B, _EMB), jnp.float32),
        scratch_types=[
            pltpu.VMEM((_EPW * _L,), jnp.int32),
        ] + [pltpu.VMEM((2 * _HALF, _EMB), jnp.float32)] * _NBUF + [
            pltpu.VMEM((_OUT_TILE, _EMB), jnp.float32),
        ] + [pltpu.SemaphoreType.DMA] * (_NBUF + 1),
    )(idx_flat, table)
    out = pl.pallas_call(
        _head_body,
        out_shape=jax.ShapeDtypeStruct((_B, _LBL), jnp.float32),
    )(pooled, W, jnp.reshape(b, (1, _LBL)))
    return out


# speedup vs baseline: 1.0037x; 1.0026x over previous
import jax
import jax.numpy as jnp
from jax import lax
from jax.experimental import pallas as pl
from jax.experimental.pallas import tpu as pltpu
from jax.experimental.pallas import tpu_sc as plsc

_NC = 2
_NS = 16
_NW = _NC * _NS
_L = 200
_B = 4096
_EMB = 128
_LBL = 100
_EPW = _B // _NW
_HALF = _L // 2
_VEC = 16
_KV = _EMB // _VEC
_NBUF = 4
_G0 = 96
_G1 = _L - _G0
_OUT_TILE = 8
_SCALE = 1.0 / _L


def _sc_pool_body(idx_hbm, table_hbm, out_hbm, idx_v,
                  rows0, rows1, rows2, rows3, outst,
                  sem0, sem1, sem2, sem3, out_sem, idx_sem):
    bufs = (rows0, rows1, rows2, rows3)
    sems = (sem0, sem1, sem2, sem3)
    wid = lax.axis_index("s") * _NC + lax.axis_index("c")
    base = pl.multiple_of(wid * _EPW, _EPW)
    nwords = _EPW * _L
    head_words = _NBUF * _L
    hbase = pl.multiple_of(wid * nwords, 8)
    pltpu.sync_copy(idx_hbm.at[pl.ds(hbase, head_words)],
                    idx_v.at[pl.ds(0, head_words)])
    rest = pltpu.async_copy(
        idx_hbm.at[pl.ds(pl.multiple_of(wid * nwords + head_words, 8),
                         nwords - head_words)],
        idx_v.at[pl.ds(head_words, nwords - head_words)], idx_sem)

    _SPLITS = ((0, _G0), (_G0, _G1))

    def issue(e, buf, sem):
        for s, n in _SPLITS:
            off = pl.multiple_of(e * _L + s, 8)
            pltpu.async_copy(table_hbm.at[idx_v.at[pl.ds(off, n)]],
                             buf.at[pl.ds(s, n)], sem)

    def drain(buf, sem):
        for s, n in _SPLITS:
            pltpu.make_async_copy(table_hbm.at[idx_v.at[pl.ds(0, n)]],
                                  buf.at[pl.ds(s, n)], sem).wait()

    def consume(e, buf):
        zeros = tuple(jnp.zeros((_VEC,), jnp.float32) for _ in range(_KV))

        @plsc.parallel_loop(0, _HALF, unroll=2, carry=zeros)
        def acc(l, acc):
            acc = list(acc)
            for h in range(2):
                for k in range(_KV):
                    acc[k] = acc[k] + buf[l + _HALF * h,
                                          pl.ds(k * _VEC, _VEC)]
            return tuple(acc)
        m = lax.rem(e, _OUT_TILE)

        @pl.when(jnp.logical_and(m == 0, e >= _OUT_TILE))
        def _drain_prev_flush():
            pltpu.make_async_copy(out_hbm.at[pl.ds(0, _OUT_TILE)], outst,
                                  out_sem).wait()

        for k in range(_KV):
            outst[m, pl.ds(k * _VEC, _VEC)] = acc[k] * _SCALE

        @pl.when(m == _OUT_TILE - 1)
        def _flush():
            start = pl.multiple_of(base + e - (_OUT_TILE - 1), _OUT_TILE)
            pltpu.async_copy(outst, out_hbm.at[pl.ds(start, _OUT_TILE)],
                             out_sem)

    for q in range(_NBUF):
        issue(q, bufs[q], sems[q])
    rest.wait()

    def step(g, carry):
        e0 = _NBUF * g
        for q in range(_NBUF):
            e = e0 + q
            drain(bufs[q], sems[q])
            consume(e, bufs[q])

            @pl.when(e + _NBUF < _EPW)
            def _():
                issue(e + _NBUF, bufs[q], sems[q])
        return carry

    lax.fori_loop(0, _EPW // _NBUF, step, jnp.int32(0))
    pltpu.make_async_copy(out_hbm.at[pl.ds(0, _OUT_TILE)], outst,
                          out_sem).wait()


def _head_body(z_ref, w_ref, b_ref, o_ref):
    o_ref[...] = jax.nn.sigmoid(
        jnp.dot(z_ref[...], w_ref[...], preferred_element_type=jnp.float32)
        + b_ref[...])


def kernel(inputs, table, W, b):
    idx_flat = jnp.transpose(inputs).reshape(-1)
    mesh = plsc.VectorSubcoreMesh(core_axis_name="c", subcore_axis_name="s")
    pooled = pl.kernel(
        _sc_pool_body,
        mesh=mesh,
        out_type=jax.ShapeDtypeStruct((_B, _EMB), jnp.float32),
        scratch_types=[
            pltpu.VMEM((_EPW * _L,), jnp.int32),
        ] + [pltpu.VMEM((2 * _HALF, _EMB), jnp.float32)] * _NBUF + [
            pltpu.VMEM((_OUT_TILE, _EMB), jnp.float32),
        ] + [pltpu.SemaphoreType.DMA] * (_NBUF + 2),
    )(idx_flat, table)
    out = pl.pallas_call(
        _head_body,
        out_shape=jax.ShapeDtypeStruct((_B, _LBL), jnp.float32),
    )(pooled, W, jnp.reshape(b, (1, _LBL)))
    return out
